# Initial kernel scaffold; baseline (speedup 1.0000x reference)
import jax
import jax.numpy as jnp
from jax.experimental import pallas as pl


def kernel(nfeats, efeats, edge_index, W_msg1, b_msg1, W_app1, b_app1, W_msg2, b_msg2, W_app2, b_app2):
    raise NotImplementedError("write your pallas kernel here")


# speedup vs baseline: 2.6470x; 2.6470x over previous
import functools

import jax
import jax.numpy as jnp
from jax import lax
from jax.experimental import pallas as pl
from jax.experimental.pallas import tpu as pltpu
from jax.experimental.pallas import tpu_sc as plsc

NC = 2
NS = 16
CH = 128


def _sc_mesh():
    return plsc.VectorSubcoreMesh(core_axis_name="c", subcore_axis_name="s")


def _segsum_cols(table, idxr, dstr, zrows, n_pad, w):
    nchunks = dstr.shape[0]
    cpt = nchunks // NS
    stripe = n_pad // NS

    @functools.partial(
        pl.kernel,
        out_type=jax.ShapeDtypeStruct((NC, n_pad, w), jnp.float32),
        mesh=_sc_mesh(),
        compiler_params=pltpu.CompilerParams(use_tc_tiling_on_sc=False),
        scratch_types=[
            pltpu.VMEM((cpt, CH), jnp.int32),
            pltpu.VMEM((cpt, CH), jnp.int32),
            pltpu.VMEM((CH, w), jnp.float32),
            pltpu.VMEM((CH, w), jnp.float32),
            pltpu.VMEM_SHARED((n_pad, w), jnp.float32),
            pltpu.SemaphoreType.DMA,
            pltpu.SemaphoreType.DMA,
        ],
    )
    def body(tab_hbm, idx_hbm, dst_hbm, zr_hbm,
             out_ss, idx_v, dst_v, bufa, bufb, accum, sema, semb):
        c = lax.axis_index("c")
        s = lax.axis_index("s")
        pltpu.sync_copy(zr_hbm, accum.at[pl.ds(s * stripe, stripe)])
        pltpu.sync_copy(idx_hbm.at[c, pl.ds(s * cpt, cpt)], idx_v)
        pltpu.sync_copy(dst_hbm.at[pl.ds(s * cpt, cpt)], dst_v)
        plsc.subcore_barrier()

        pltpu.async_copy(tab_hbm.at[idx_v.at[0]], bufa, sema)
        pltpu.async_copy(tab_hbm.at[idx_v.at[1]], bufb, semb)

        def main_body(jj, _):
            j0 = jj * 2
            j1 = j0 + 1
            pltpu.make_async_copy(tab_hbm.at[idx_v.at[j0]], bufa, sema).wait()
            pltpu.sync_copy(bufa, accum.at[dst_v.at[j0]], add=True)

            @pl.when(j0 + 2 < cpt)
            def _():
                pltpu.async_copy(tab_hbm.at[idx_v.at[j0 + 2]], bufa, sema)

            pltpu.make_async_copy(tab_hbm.at[idx_v.at[j1]], bufb, semb).wait()
            pltpu.sync_copy(bufb, accum.at[dst_v.at[j1]], add=True)

            @pl.when(j1 + 2 < cpt)
            def _():
                pltpu.async_copy(tab_hbm.at[idx_v.at[j1 + 2]], bufb, semb)

        lax.fori_loop(0, cpt // 2, main_body, None)

        plsc.subcore_barrier()
        pltpu.sync_copy(accum.at[pl.ds(s * stripe, stripe)],
                        out_ss.at[c, pl.ds(s * stripe, stripe)])

    return body(table, idxr, dstr, zrows)


def _segsum_aug(efr, dstr, zaug, n_pad, d_aug):
    nchunks = dstr.shape[0]
    cpw = nchunks // (NC * NS)
    stripe = n_pad // NS

    @functools.partial(
        pl.kernel,
        out_type=jax.ShapeDtypeStruct((NC, n_pad, d_aug), jnp.float32),
        mesh=_sc_mesh(),
        compiler_params=pltpu.CompilerParams(use_tc_tiling_on_sc=False),
        scratch_types=[
            pltpu.VMEM((cpw, CH), jnp.int32),
            pltpu.VMEM((CH, d_aug), jnp.float32),
            pltpu.VMEM((CH, d_aug), jnp.float32),
            pltpu.VMEM_SHARED((n_pad, d_aug), jnp.float32),
            pltpu.SemaphoreType.DMA,
            pltpu.SemaphoreType.DMA,
        ],
    )
    def body(ef_hbm, dst_hbm, za_hbm, out_aug,
             dst_v, bufa, bufb, aug_acc, sema, semb):
        c = lax.axis_index("c")
        s = lax.axis_index("s")
        wid = s * NC + c
        pltpu.sync_copy(za_hbm, aug_acc.at[pl.ds(s * stripe, stripe)])
        pltpu.sync_copy(dst_hbm.at[pl.ds(wid * cpw, cpw)], dst_v)
        plsc.subcore_barrier()

        pltpu.async_copy(ef_hbm.at[wid * cpw], bufa, sema)
        pltpu.async_copy(ef_hbm.at[wid * cpw + 1], bufb, semb)

        def main_body(jj, _):
            j0 = jj * 2
            j1 = j0 + 1
            pltpu.make_async_copy(ef_hbm.at[wid * cpw + j0], bufa, sema).wait()
            pltpu.sync_copy(bufa, aug_acc.at[dst_v.at[j0]], add=True)

            @pl.when(j0 + 2 < cpw)
            def _():
                pltpu.async_copy(ef_hbm.at[wid * cpw + j0 + 2], bufa, sema)

            pltpu.make_async_copy(ef_hbm.at[wid * cpw + j1], bufb, semb).wait()
            pltpu.sync_copy(bufb, aug_acc.at[dst_v.at[j1]], add=True)

            @pl.when(j1 + 2 < cpw)
            def _():
                pltpu.async_copy(ef_hbm.at[wid * cpw + j1 + 2], bufb, semb)

        lax.fori_loop(0, cpw // 2, main_body, None)

        plsc.subcore_barrier()
        pltpu.sync_copy(aug_acc.at[pl.ds(s * stripe, stripe)],
                        out_aug.at[c, pl.ds(s * stripe, stripe)])

    return body(efr, dstr, zaug)


def _l1_combine_kernel(nf_ref, ssa_ref, ssb_ref, aug_ref, wm_ref, bm_ref,
                       wa_ref, ba_ref, out_ref):
    nf = nf_ref[...]
    aug = aug_ref[0] + aug_ref[1]
    wm = wm_ref[...]
    d_in = nf.shape[1]
    q = d_in // 4
    cnt = aug[:, 16]
    sege = aug[:, :16]
    inv = 1.0 / jnp.maximum(cnt, 1.0)
    raw = (jnp.dot(ssa_ref[0], wm[:q], preferred_element_type=jnp.float32)
           + jnp.dot(ssa_ref[1], wm[q:2 * q],
                     preferred_element_type=jnp.float32)
           + jnp.dot(ssb_ref[0], wm[2 * q:3 * q],
                     preferred_element_type=jnp.float32)
           + jnp.dot(ssb_ref[1], wm[3 * q:4 * q],
                     preferred_element_type=jnp.float32)
           + jnp.dot(sege, wm[d_in:], preferred_element_type=jnp.float32)
           + cnt[:, None] * bm_ref[...])
    hn = raw * inv[:, None]
    wa = wa_ref[...]
    h1 = (jnp.dot(nf, wa[:d_in], preferred_element_type=jnp.float32)
          + jnp.dot(hn, wa[d_in:], preferred_element_type=jnp.float32)
          + ba_ref[...])
    out_ref[...] = jnp.maximum(h1, 0.0)


def _l2_combine_kernel(h1_ref, ss_ref, aug_ref, wm_ref, bm_ref, wa_ref,
                       ba_ref, out_ref):
    h1 = h1_ref[...]
    aug = aug_ref[0] + aug_ref[1]
    wm = wm_ref[...]
    dh = h1.shape[1]
    q = dh // 2
    cnt = aug[:, 16]
    sege = aug[:, :16]
    inv = 1.0 / jnp.maximum(cnt, 1.0)
    raw = (jnp.dot(ss_ref[0], wm[:q], preferred_element_type=jnp.float32)
           + jnp.dot(ss_ref[1], wm[q:dh], preferred_element_type=jnp.float32)
           + jnp.dot(sege, wm[dh:], preferred_element_type=jnp.float32)
           + cnt[:, None] * bm_ref[...])
    hn = raw * inv[:, None]
    wa = wa_ref[...]
    out = (jnp.dot(h1, wa[:dh], preferred_element_type=jnp.float32)
           + jnp.dot(hn, wa[dh:], preferred_element_type=jnp.float32)
           + ba_ref[...])
    out_ref[...] = jnp.maximum(out, 0.0)


def kernel(nfeats, efeats, edge_index, W_msg1, b_msg1, W_app1, b_app1,
           W_msg2, b_msg2, W_app2, b_app2):
    N, d_in = nfeats.shape
    E, d_edge = efeats.shape
    d_hid = W_app1.shape[1]
    d_out = W_app2.shape[1]
    d_aug = 32
    W = 64

    epad = -E % (NC * NS * CH)
    e_tot = E + epad
    n_pad = pl.cdiv(N + 1, NS * 8) * NS * 8

    src = edge_index[0].astype(jnp.int32)
    dst = edge_index[1].astype(jnp.int32)
    src = jnp.concatenate([src, jnp.zeros((epad,), jnp.int32)])
    dst = jnp.concatenate([dst, jnp.full((epad,), N, jnp.int32)])
    nchunks = e_tot // CH
    srcr = src.reshape(nchunks, CH)
    dstr = dst.reshape(nchunks, CH)
    src4 = srcr * 4
    src2 = srcr * 2
    nf_q = nfeats.reshape(N * (d_in // W), W)

    ef_aug = jnp.concatenate(
        [efeats, jnp.ones((E, 1), jnp.float32),
         jnp.zeros((E, d_aug - d_edge - 1), jnp.float32)], axis=1)
    efr = jnp.concatenate(
        [ef_aug, jnp.zeros((epad, d_aug), jnp.float32)]).reshape(
            nchunks, CH, d_aug)

    stripe = n_pad // NS
    zrows = jnp.zeros((stripe, W), jnp.float32)
    zaug = jnp.zeros((stripe, d_aug), jnp.float32)

    ss1a = _segsum_cols(nf_q, jnp.stack([src4, src4 + 1]), dstr, zrows,
                        n_pad, W)
    ss1b = _segsum_cols(nf_q, jnp.stack([src4 + 2, src4 + 3]), dstr, zrows,
                        n_pad, W)
    aug = _segsum_aug(efr, dstr, zaug, n_pad, d_aug)

    R = 1000
    grid = (N // R,)
    wspec = lambda shape: pl.BlockSpec(shape, lambda i: (0,) * len(shape))
    h1 = pl.pallas_call(
        _l1_combine_kernel,
        grid=grid,
        in_specs=[
            pl.BlockSpec((R, d_in), lambda i: (i, 0)),
            pl.BlockSpec((NC, R, W), lambda i: (0, i, 0)),
            pl.BlockSpec((NC, R, W), lambda i: (0, i, 0)),
            pl.BlockSpec((NC, R, d_aug), lambda i: (0, i, 0)),
            wspec((d_in + d_edge, d_hid)),
            wspec((1, d_hid)),
            wspec((d_in + d_hid, d_hid)),
            wspec((1, d_hid)),
        ],
        out_specs=pl.BlockSpec((R, d_hid), lambda i: (i, 0)),
        out_shape=jax.ShapeDtypeStruct((N, d_hid), jnp.float32),
    )(nfeats, ss1a[:, :N], ss1b[:, :N], aug[:, :N], W_msg1,
      b_msg1.reshape(1, -1), W_app1, b_app1.reshape(1, -1))

    h1_q = h1.reshape(N * (d_hid // W), W)
    ss2 = _segsum_cols(h1_q, jnp.stack([src2, src2 + 1]), dstr, zrows,
                       n_pad, W)

    out = pl.pallas_call(
        _l2_combine_kernel,
        grid=grid,
        in_specs=[
            pl.BlockSpec((R, d_hid), lambda i: (i, 0)),
            pl.BlockSpec((NC, R, W), lambda i: (0, i, 0)),
            pl.BlockSpec((NC, R, d_aug), lambda i: (0, i, 0)),
            wspec((d_hid + d_edge, d_out)),
            wspec((1, d_out)),
            wspec((d_hid + d_out, d_out)),
            wspec((1, d_out)),
        ],
        out_specs=pl.BlockSpec((R, d_out), lambda i: (i, 0)),
        out_shape=jax.ShapeDtypeStruct((N, d_out), jnp.float32),
    )(h1, ss2[:, :N], aug[:, :N], W_msg2, b_msg2.reshape(1, -1),
      W_app2, b_app2.reshape(1, -1))
    return out
